# baseline (speedup 1.0000x reference)
import jax
import jax.numpy as jnp
from jax import lax
from jax.experimental import pallas as pl
from jax.experimental.pallas import tpu as pltpu
from jax.experimental.pallas import tpu_sc as plsc

N_NODES = 100000
H = 128

NC = 2
NS = 16
NW = NC * NS
CHUNK = 64
N_CHUNKS = 1600
N_PAD = CHUNK * N_CHUNKS
SPLIT0 = 76
SPLIT1 = 24


def _sc_gather_body(t_hbm, i0a_hbm, i1a_hbm, i0b_hbm, i1b_hbm,
                    og0, og1,
                    i0v, i1v, bufs, semga, semgb, semsa, semsb):
    cid = lax.axis_index("c")
    sid = lax.axis_index("s")

    outs = (og0, og1)

    def drain_stores(s, sem):
        for t in range(2):
            pltpu.make_async_copy(
                bufs.at[s, t], outs[t].at[pl.ds(0, CHUNK)], sem).wait()

    def fire_gathers(s, j, sem):
        return [
            pltpu.async_copy(t_hbm.at[i0v.at[j]], bufs.at[s, 0], sem),
            pltpu.async_copy(t_hbm.at[i1v.at[j]], bufs.at[s, 1], sem),
        ]

    def fire_stores(s, j, chunk_base, sem):
        base = (chunk_base + j) * CHUNK
        for t in range(2):
            pltpu.async_copy(bufs.at[s, t], outs[t].at[pl.ds(base, CHUNK)],
                             sem)

    def run(i0slab, i1slab, nch, chunk_base):
        pltpu.sync_copy(i0slab.at[sid], i0v.at[pl.ds(0, nch)])
        pltpu.sync_copy(i1slab.at[sid], i1v.at[pl.ds(0, nch)])

        def pair(p, carry):
            ja = 2 * p
            jb = 2 * p + 1
            pl.when(p > 0)(lambda: drain_stores(0, semsa))
            ga = fire_gathers(0, ja, semga)
            pl.when(p > 0)(lambda: drain_stores(1, semsb))
            gb = fire_gathers(1, jb, semgb)
            for g in ga:
                g.wait()
            fire_stores(0, ja, chunk_base, semsa)
            for g in gb:
                g.wait()
            fire_stores(1, jb, chunk_base, semsb)
            return carry

        lax.fori_loop(0, nch // 2, pair, 0)
        drain_stores(0, semsa)
        drain_stores(1, semsb)

    if SPLIT0 > 0:
        pl.when(cid == 0)(
            lambda: run(i0a_hbm, i1a_hbm, SPLIT0, sid * SPLIT0))
    if SPLIT1 > 0:
        pl.when(cid == 1)(
            lambda: run(i0b_hbm, i1b_hbm, SPLIT1, NS * SPLIT0 + sid * SPLIT1))


@jax.jit
def _sc_gather(t, i0a, i1a, i0b, i1b):
    mesh = plsc.VectorSubcoreMesh(core_axis_name="c", subcore_axis_name="s")
    row = jax.ShapeDtypeStruct((N_PAD, H), jnp.int32)
    fn = pl.kernel(
        _sc_gather_body,
        mesh=mesh,
        out_type=(row, row),
        scratch_types=[
            pltpu.VMEM((max(SPLIT0, SPLIT1), CHUNK), jnp.int32),
            pltpu.VMEM((max(SPLIT0, SPLIT1), CHUNK), jnp.int32),
            pltpu.VMEM((2, 2, CHUNK, H), jnp.int32),
            pltpu.SemaphoreType.DMA,
            pltpu.SemaphoreType.DMA,
            pltpu.SemaphoreType.DMA,
            pltpu.SemaphoreType.DMA,
        ],
    )
    return fn(t, i0a, i1a, i0b, i1b)


PBLK = 2048


def _pack_body(h_ref, c_ref, t_ref):
    def tobf(u):
        return (u + jnp.uint32(0x7FFF) + ((u >> 16) & jnp.uint32(1))) >> 16

    def packhalves(u):
        lo = tobf(u[:, :64])
        hi = tobf(u[:, 64:])
        return lo | (hi << 16)

    uh = lax.bitcast_convert_type(h_ref[...], jnp.uint32)
    uc = lax.bitcast_convert_type(c_ref[...], jnp.uint32)
    t = jnp.concatenate([packhalves(uh), packhalves(uc)], axis=1)
    t_ref[...] = lax.bitcast_convert_type(t, jnp.int32)


@jax.jit
def _pack(h, c):
    n = h.shape[0]
    return pl.pallas_call(
        _pack_body,
        grid=(pl.cdiv(n, PBLK),),
        in_specs=[
            pl.BlockSpec((PBLK, H), lambda i: (i, 0)),
            pl.BlockSpec((PBLK, H), lambda i: (i, 0)),
        ],
        out_specs=pl.BlockSpec((PBLK, H), lambda i: (i, 0)),
        out_shape=jax.ShapeDtypeStruct((n, H), jnp.int32),
        compiler_params=pltpu.CompilerParams(
            dimension_semantics=("arbitrary",),
        ),
    )(h, c)


BLK = 1024


def _unpack(g_ref):
    u = lax.bitcast_convert_type(g_ref[...], jnp.uint32)

    def expand(half):
        lo = lax.bitcast_convert_type(half << 16, jnp.float32)
        hi = lax.bitcast_convert_type(half & jnp.uint32(0xFFFF0000),
                                      jnp.float32)
        return jnp.concatenate([lo, hi], axis=1)

    return expand(u[:, :64]), expand(u[:, 64:])


def _dense_body(x_ref, g0_ref, g1_ref,
                w_ref, u0_ref, u1_ref, b_ref, f0_ref, f1_ref, bf_ref,
                hout_ref, cout_ref):
    x = x_ref[...]
    h0, c0 = _unpack(g0_ref)
    h1, c1 = _unpack(g1_ref)
    f32 = jnp.float32
    iou = (jnp.dot(x, w_ref[...], preferred_element_type=f32)
           + jnp.dot(h0, u0_ref[...], preferred_element_type=f32)
           + jnp.dot(h1, u1_ref[...], preferred_element_type=f32)
           + b_ref[...])
    fpre = (jnp.dot(h0, f0_ref[...], preferred_element_type=f32)
            + jnp.dot(h1, f1_ref[...], preferred_element_type=f32)
            + bf_ref[...])
    f = jax.nn.sigmoid(fpre)
    c_agg = f[:, :H] * c0 + f[:, H:] * c1
    i = jax.nn.sigmoid(iou[:, :H])
    o = jax.nn.sigmoid(iou[:, H:2 * H])
    u = jnp.tanh(iou[:, 2 * H:])
    c_new = i * u + c_agg
    hout_ref[...] = o * jnp.tanh(c_new)
    cout_ref[...] = c_new


@jax.jit
def _dense(x, g0, g1, W_iou, Um0, Um1, b_iou, Uf0, Uf1, bf):
    n = x.shape[0]
    grid = (pl.cdiv(n, BLK),)
    row_spec = pl.BlockSpec((BLK, H), lambda i: (i, 0))
    full = lambda s: pl.BlockSpec(s, lambda i: (0, 0))
    return pl.pallas_call(
        _dense_body,
        grid=grid,
        in_specs=[
            row_spec, row_spec, row_spec,
            full((H, 3 * H)), full((H, 3 * H)), full((H, 3 * H)),
            full((1, 3 * H)),
            full((H, 2 * H)), full((H, 2 * H)), full((1, 2 * H)),
        ],
        out_specs=[
            pl.BlockSpec((BLK, H), lambda i: (i, 0)),
            pl.BlockSpec((BLK, H), lambda i: (i, 0)),
        ],
        out_shape=[
            jax.ShapeDtypeStruct((n, H), jnp.float32),
            jax.ShapeDtypeStruct((n, H), jnp.float32),
        ],
        compiler_params=pltpu.CompilerParams(
            dimension_semantics=("arbitrary",),
        ),
    )(x, g0, g1, W_iou, Um0, Um1, b_iou, Uf0, Uf1, bf)


def kernel(x, h, c, child_idx, W_iou, Um0_iou, Um1_iou, b_iou, U_f_w, U_f_b):
    idx = child_idx.astype(jnp.int32)
    pad = N_PAD - N_NODES
    na = NS * SPLIT0 * CHUNK
    idx0 = jnp.pad(idx[:, 0], (0, pad))
    idx1 = jnp.pad(idx[:, 1], (0, pad))
    if SPLIT0 > 0:
        i0a = idx0[:na].reshape(NS, SPLIT0, CHUNK)
        i1a = idx1[:na].reshape(NS, SPLIT0, CHUNK)
    else:
        i0a = i1a = idx0[:NS * CHUNK].reshape(NS, 1, CHUNK)
    if SPLIT1 > 0:
        i0b = idx0[na:].reshape(NS, SPLIT1, CHUNK)
        i1b = idx1[na:].reshape(NS, SPLIT1, CHUNK)
    else:
        i0b = i1b = idx0[:NS * CHUNK].reshape(NS, 1, CHUNK)
    t = _pack(h, c)
    g0, g1 = _sc_gather(t, i0a, i1a, i0b, i1b)
    bf = U_f_b.reshape(1, 2 * H)
    h_new, c_new = _dense(x, g0, g1,
                          W_iou, Um0_iou, Um1_iou, b_iou,
                          U_f_w[:H, :], U_f_w[H:, :], bf)
    return h_new, c_new


# speedup vs baseline: 1.0146x; 1.0146x over previous
import jax
import jax.numpy as jnp
from jax import lax
from jax.experimental import pallas as pl
from jax.experimental.pallas import tpu as pltpu
from jax.experimental.pallas import tpu_sc as plsc

N_NODES = 100000
H = 128

NC = 2
NS = 16
NW = NC * NS
CHUNK = 64
N_CHUNKS = 1600
N_PAD = CHUNK * N_CHUNKS
SPLIT0 = 52
SPLIT1 = 48
NSETS = 4


def _sc_gather_body(t_hbm, i0a_hbm, i1a_hbm, i0b_hbm, i1b_hbm,
                    og0, og1,
                    i0v, i1v, bufs, semg, sems):
    cid = lax.axis_index("c")
    sid = lax.axis_index("s")

    outs = (og0, og1)

    def drain_stores(s):
        for t in range(2):
            pltpu.make_async_copy(
                bufs.at[s, t], outs[t].at[pl.ds(0, CHUNK)],
                sems.at[s]).wait()

    def fire_gathers(s, j):
        return [
            pltpu.async_copy(t_hbm.at[i0v.at[j]], bufs.at[s, 0],
                             semg.at[s]),
            pltpu.async_copy(t_hbm.at[i1v.at[j]], bufs.at[s, 1],
                             semg.at[s]),
        ]

    def fire_stores(s, j, chunk_base):
        base = (chunk_base + j) * CHUNK
        for t in range(2):
            pltpu.async_copy(bufs.at[s, t], outs[t].at[pl.ds(base, CHUNK)],
                             sems.at[s])

    def run(i0slab, i1slab, nch, chunk_base):
        pltpu.sync_copy(i0slab.at[sid], i0v.at[pl.ds(0, nch)])
        pltpu.sync_copy(i1slab.at[sid], i1v.at[pl.ds(0, nch)])

        def quad(q, carry):
            gs = []
            for s in range(NSETS):
                j = NSETS * q + s
                pl.when(q > 0)(lambda s=s: drain_stores(s))
                gs.append(fire_gathers(s, j))
            for s in range(NSETS):
                j = NSETS * q + s
                for g in gs[s]:
                    g.wait()
                fire_stores(s, j, chunk_base)
            return carry

        lax.fori_loop(0, nch // NSETS, quad, 0)
        for s in range(NSETS):
            drain_stores(s)

    if SPLIT0 > 0:
        pl.when(cid == 0)(
            lambda: run(i0a_hbm, i1a_hbm, SPLIT0, sid * SPLIT0))
    if SPLIT1 > 0:
        pl.when(cid == 1)(
            lambda: run(i0b_hbm, i1b_hbm, SPLIT1, NS * SPLIT0 + sid * SPLIT1))


@jax.jit
def _sc_gather(t, i0a, i1a, i0b, i1b):
    mesh = plsc.VectorSubcoreMesh(core_axis_name="c", subcore_axis_name="s")
    row = jax.ShapeDtypeStruct((N_PAD, H), jnp.int32)
    fn = pl.kernel(
        _sc_gather_body,
        mesh=mesh,
        out_type=(row, row),
        scratch_types=[
            pltpu.VMEM((max(SPLIT0, SPLIT1), CHUNK), jnp.int32),
            pltpu.VMEM((max(SPLIT0, SPLIT1), CHUNK), jnp.int32),
            pltpu.VMEM((NSETS, 2, CHUNK, H), jnp.int32),
            pltpu.SemaphoreType.DMA((NSETS,)),
            pltpu.SemaphoreType.DMA((NSETS,)),
        ],
    )
    return fn(t, i0a, i1a, i0b, i1b)


PBLK = 2048


def _pack_body(h_ref, c_ref, t_ref):
    def tobf(u):
        return (u + jnp.uint32(0x7FFF) + ((u >> 16) & jnp.uint32(1))) >> 16

    def packhalves(u):
        lo = tobf(u[:, :64])
        hi = tobf(u[:, 64:])
        return lo | (hi << 16)

    uh = lax.bitcast_convert_type(h_ref[...], jnp.uint32)
    uc = lax.bitcast_convert_type(c_ref[...], jnp.uint32)
    t = jnp.concatenate([packhalves(uh), packhalves(uc)], axis=1)
    t_ref[...] = lax.bitcast_convert_type(t, jnp.int32)


@jax.jit
def _pack(h, c):
    n = h.shape[0]
    return pl.pallas_call(
        _pack_body,
        grid=(pl.cdiv(n, PBLK),),
        in_specs=[
            pl.BlockSpec((PBLK, H), lambda i: (i, 0)),
            pl.BlockSpec((PBLK, H), lambda i: (i, 0)),
        ],
        out_specs=pl.BlockSpec((PBLK, H), lambda i: (i, 0)),
        out_shape=jax.ShapeDtypeStruct((n, H), jnp.int32),
        compiler_params=pltpu.CompilerParams(
            dimension_semantics=("arbitrary",),
        ),
    )(h, c)


BLK = 1024


def _unpack(g_ref):
    u = lax.bitcast_convert_type(g_ref[...], jnp.uint32)

    def expand(half):
        lo = lax.bitcast_convert_type(half << 16, jnp.float32)
        hi = lax.bitcast_convert_type(half & jnp.uint32(0xFFFF0000),
                                      jnp.float32)
        return jnp.concatenate([lo, hi], axis=1)

    return expand(u[:, :64]), expand(u[:, 64:])


def _dense_body(x_ref, g0_ref, g1_ref,
                w_ref, u0_ref, u1_ref, b_ref, f0_ref, f1_ref, bf_ref,
                hout_ref, cout_ref):
    x = x_ref[...]
    h0, c0 = _unpack(g0_ref)
    h1, c1 = _unpack(g1_ref)
    f32 = jnp.float32
    iou = (jnp.dot(x, w_ref[...], preferred_element_type=f32)
           + jnp.dot(h0, u0_ref[...], preferred_element_type=f32)
           + jnp.dot(h1, u1_ref[...], preferred_element_type=f32)
           + b_ref[...])
    fpre = (jnp.dot(h0, f0_ref[...], preferred_element_type=f32)
            + jnp.dot(h1, f1_ref[...], preferred_element_type=f32)
            + bf_ref[...])
    f = jax.nn.sigmoid(fpre)
    c_agg = f[:, :H] * c0 + f[:, H:] * c1
    i = jax.nn.sigmoid(iou[:, :H])
    o = jax.nn.sigmoid(iou[:, H:2 * H])
    u = jnp.tanh(iou[:, 2 * H:])
    c_new = i * u + c_agg
    hout_ref[...] = o * jnp.tanh(c_new)
    cout_ref[...] = c_new


@jax.jit
def _dense(x, g0, g1, W_iou, Um0, Um1, b_iou, Uf0, Uf1, bf):
    n = x.shape[0]
    grid = (pl.cdiv(n, BLK),)
    row_spec = pl.BlockSpec((BLK, H), lambda i: (i, 0))
    full = lambda s: pl.BlockSpec(s, lambda i: (0, 0))
    return pl.pallas_call(
        _dense_body,
        grid=grid,
        in_specs=[
            row_spec, row_spec, row_spec,
            full((H, 3 * H)), full((H, 3 * H)), full((H, 3 * H)),
            full((1, 3 * H)),
            full((H, 2 * H)), full((H, 2 * H)), full((1, 2 * H)),
        ],
        out_specs=[
            pl.BlockSpec((BLK, H), lambda i: (i, 0)),
            pl.BlockSpec((BLK, H), lambda i: (i, 0)),
        ],
        out_shape=[
            jax.ShapeDtypeStruct((n, H), jnp.float32),
            jax.ShapeDtypeStruct((n, H), jnp.float32),
        ],
        compiler_params=pltpu.CompilerParams(
            dimension_semantics=("arbitrary",),
        ),
    )(x, g0, g1, W_iou, Um0, Um1, b_iou, Uf0, Uf1, bf)


def kernel(x, h, c, child_idx, W_iou, Um0_iou, Um1_iou, b_iou, U_f_w, U_f_b):
    idx = child_idx.astype(jnp.int32)
    pad = N_PAD - N_NODES
    na = NS * SPLIT0 * CHUNK
    idx0 = jnp.pad(idx[:, 0], (0, pad))
    idx1 = jnp.pad(idx[:, 1], (0, pad))
    if SPLIT0 > 0:
        i0a = idx0[:na].reshape(NS, SPLIT0, CHUNK)
        i1a = idx1[:na].reshape(NS, SPLIT0, CHUNK)
    else:
        i0a = i1a = idx0[:NS * CHUNK].reshape(NS, 1, CHUNK)
    if SPLIT1 > 0:
        i0b = idx0[na:].reshape(NS, SPLIT1, CHUNK)
        i1b = idx1[na:].reshape(NS, SPLIT1, CHUNK)
    else:
        i0b = i1b = idx0[:NS * CHUNK].reshape(NS, 1, CHUNK)
    t = _pack(h, c)
    g0, g1 = _sc_gather(t, i0a, i1a, i0b, i1b)
    bf = U_f_b.reshape(1, 2 * H)
    h_new, c_new = _dense(x, g0, g1,
                          W_iou, Um0_iou, Um1_iou, b_iou,
                          U_f_w[:H, :], U_f_w[H:, :], bf)
    return h_new, c_new


# speedup vs baseline: 1.2332x; 1.2154x over previous
import jax
import jax.numpy as jnp
from jax import lax
from jax.experimental import pallas as pl
from jax.experimental.pallas import tpu as pltpu
from jax.experimental.pallas import tpu_sc as plsc

N_NODES = 100000
H = 128

NC = 2
NS = 16
NW = NC * NS
CHUNK = 64
N_CHUNKS = 1600
N_PAD = CHUNK * N_CHUNKS
NSETS = 2
SLICES = (512, 576, 512)
BLK = 1024


def _sc_gather_body(npw, t_hbm, i0_hbm, i1_hbm, og0, og1,
                    i0v, i1v, bufs, semg, sems):
    cid = lax.axis_index("c")
    sid = lax.axis_index("s")
    wid = sid * NC + cid

    outs = (og0, og1)

    def drain_stores(s):
        for t in range(2):
            pltpu.make_async_copy(
                bufs.at[s, t], outs[t].at[pl.ds(0, CHUNK)],
                sems.at[s]).wait()

    def fire_gathers(s, j):
        return [
            pltpu.async_copy(t_hbm.at[i0v.at[j]], bufs.at[s, 0],
                             semg.at[s]),
            pltpu.async_copy(t_hbm.at[i1v.at[j]], bufs.at[s, 1],
                             semg.at[s]),
        ]

    def fire_stores(s, j):
        base = (wid * npw + j) * CHUNK
        for t in range(2):
            pltpu.async_copy(bufs.at[s, t], outs[t].at[pl.ds(base, CHUNK)],
                             sems.at[s])

    pltpu.sync_copy(i0_hbm.at[wid], i0v)
    pltpu.sync_copy(i1_hbm.at[wid], i1v)

    def group(q, carry):
        gs = []
        for s in range(NSETS):
            j = NSETS * q + s
            pl.when(q > 0)(lambda s=s: drain_stores(s))
            gs.append(fire_gathers(s, j))
        for s in range(NSETS):
            j = NSETS * q + s
            for g in gs[s]:
                g.wait()
            fire_stores(s, j)
        return carry

    lax.fori_loop(0, npw // NSETS, group, 0)
    for s in range(NSETS):
        drain_stores(s)


def _make_sc_gather(n_chunks):
    npw = n_chunks // NW
    assert npw * NW == n_chunks and npw % NSETS == 0
    mesh = plsc.VectorSubcoreMesh(core_axis_name="c", subcore_axis_name="s")
    row = jax.ShapeDtypeStruct((n_chunks * CHUNK, H), jnp.int32)

    def body(*args):
        return _sc_gather_body(npw, *args)

    return pl.kernel(
        body,
        mesh=mesh,
        out_type=(row, row),
        scratch_types=[
            pltpu.VMEM((npw, CHUNK), jnp.int32),
            pltpu.VMEM((npw, CHUNK), jnp.int32),
            pltpu.VMEM((NSETS, 2, CHUNK, H), jnp.int32),
            pltpu.SemaphoreType.DMA((NSETS,)),
            pltpu.SemaphoreType.DMA((NSETS,)),
        ],
    )


PBLK = 2048


def _pack_body(h_ref, c_ref, t_ref):
    def tobf(u):
        return (u + jnp.uint32(0x7FFF) + ((u >> 16) & jnp.uint32(1))) >> 16

    def packhalves(u):
        lo = tobf(u[:, :64])
        hi = tobf(u[:, 64:])
        return lo | (hi << 16)

    uh = lax.bitcast_convert_type(h_ref[...], jnp.uint32)
    uc = lax.bitcast_convert_type(c_ref[...], jnp.uint32)
    t = jnp.concatenate([packhalves(uh), packhalves(uc)], axis=1)
    t_ref[...] = lax.bitcast_convert_type(t, jnp.int32)


def _pack(h, c):
    n = h.shape[0]
    return pl.pallas_call(
        _pack_body,
        grid=(pl.cdiv(n, PBLK),),
        in_specs=[
            pl.BlockSpec((PBLK, H), lambda i: (i, 0)),
            pl.BlockSpec((PBLK, H), lambda i: (i, 0)),
        ],
        out_specs=pl.BlockSpec((PBLK, H), lambda i: (i, 0)),
        out_shape=jax.ShapeDtypeStruct((n, H), jnp.int32),
        compiler_params=pltpu.CompilerParams(
            dimension_semantics=("arbitrary",),
        ),
    )(h, c)


def _unpack(g_ref):
    u = lax.bitcast_convert_type(g_ref[...], jnp.uint32)

    def expand(half):
        lo = lax.bitcast_convert_type(half << 16, jnp.float32)
        hi = lax.bitcast_convert_type(half & jnp.uint32(0xFFFF0000),
                                      jnp.float32)
        return jnp.concatenate([lo, hi], axis=1)

    return expand(u[:, :64]), expand(u[:, 64:])


def _dense_body(x_ref, g0_ref, g1_ref,
                w_ref, u0_ref, u1_ref, b_ref, f0_ref, f1_ref, bf_ref,
                *refs):
    hout_ref, cout_ref = refs[-2:]
    x = x_ref[...]
    h0, c0 = _unpack(g0_ref)
    h1, c1 = _unpack(g1_ref)
    f32 = jnp.float32
    iou = (jnp.dot(x, w_ref[...], preferred_element_type=f32)
           + jnp.dot(h0, u0_ref[...], preferred_element_type=f32)
           + jnp.dot(h1, u1_ref[...], preferred_element_type=f32)
           + b_ref[...])
    fpre = (jnp.dot(h0, f0_ref[...], preferred_element_type=f32)
            + jnp.dot(h1, f1_ref[...], preferred_element_type=f32)
            + bf_ref[...])
    f = jax.nn.sigmoid(fpre)
    c_agg = f[:, :H] * c0 + f[:, H:] * c1
    i = jax.nn.sigmoid(iou[:, :H])
    o = jax.nn.sigmoid(iou[:, H:2 * H])
    u = jnp.tanh(iou[:, 2 * H:])
    c_new = i * u + c_agg
    hout_ref[...] = o * jnp.tanh(c_new)
    cout_ref[...] = c_new


def _dense_slice(x, g0, g1, weights, h_prev, c_prev, blk_off, n_blocks):
    W_iou, Um0, Um1, b_iou, Uf0, Uf1, bf = weights
    n = x.shape[0]
    full = lambda s: pl.BlockSpec(s, lambda i: (0, 0))
    off_spec = pl.BlockSpec((BLK, H), lambda i: (i + blk_off, 0))
    loc_spec = pl.BlockSpec((BLK, H), lambda i: (i, 0))
    in_specs = [
        off_spec, loc_spec, loc_spec,
        full((H, 3 * H)), full((H, 3 * H)), full((H, 3 * H)),
        full((1, 3 * H)),
        full((H, 2 * H)), full((H, 2 * H)), full((1, 2 * H)),
    ]
    args = [x, g0, g1, W_iou, Um0, Um1, b_iou, Uf0, Uf1, bf]
    aliases = {}
    if h_prev is not None:
        in_specs += [pl.BlockSpec(memory_space=pltpu.MemorySpace.HBM),
                     pl.BlockSpec(memory_space=pltpu.MemorySpace.HBM)]
        args += [h_prev, c_prev]
        aliases = {10: 0, 11: 1}
    return pl.pallas_call(
        _dense_body,
        grid=(n_blocks,),
        in_specs=in_specs,
        out_specs=[
            pl.BlockSpec((BLK, H), lambda i: (i + blk_off, 0)),
            pl.BlockSpec((BLK, H), lambda i: (i + blk_off, 0)),
        ],
        out_shape=[
            jax.ShapeDtypeStruct((n, H), jnp.float32),
            jax.ShapeDtypeStruct((n, H), jnp.float32),
        ],
        input_output_aliases=aliases,
        compiler_params=pltpu.CompilerParams(
            dimension_semantics=("arbitrary",),
        ),
    )(*args)


@jax.jit
def _run(x, h, c, idx0, idx1, W_iou, Um0, Um1, b_iou, Uf0, Uf1, bf):
    t = _pack(h, c)
    weights = (W_iou, Um0, Um1, b_iou, Uf0, Uf1, bf)
    n = x.shape[0]

    gathers = []
    off = 0
    for ns in SLICES:
        npw = ns // NW
        i0 = lax.dynamic_slice_in_dim(idx0, off * CHUNK, ns * CHUNK)
        i1 = lax.dynamic_slice_in_dim(idx1, off * CHUNK, ns * CHUNK)
        i0 = i0.reshape(NW, npw, CHUNK)
        i1 = i1.reshape(NW, npw, CHUNK)
        gathers.append(_make_sc_gather(ns)(t, i0, i1))
        off += ns

    h_out = c_out = None
    off = 0
    for k, ns in enumerate(SLICES):
        g0, g1 = gathers[k]
        lo = off * CHUNK
        hi = min((off + ns) * CHUNK, n)
        n_blocks = pl.cdiv(hi - lo, BLK)
        h_out, c_out = _dense_slice(x, g0, g1, weights, h_out, c_out,
                                    lo // BLK, n_blocks)
        off += ns
    return h_out, c_out


def kernel(x, h, c, child_idx, W_iou, Um0_iou, Um1_iou, b_iou, U_f_w, U_f_b):
    idx = child_idx.astype(jnp.int32)
    pad = N_PAD - N_NODES
    idx0 = jnp.pad(idx[:, 0], (0, pad))
    idx1 = jnp.pad(idx[:, 1], (0, pad))
    bf = U_f_b.reshape(1, 2 * H)
    return _run(x, h, c, idx0, idx1,
                W_iou, Um0_iou, Um1_iou, b_iou,
                U_f_w[:H, :], U_f_w[H:, :], bf)


# speedup vs baseline: 1.2813x; 1.0390x over previous
import jax
import jax.numpy as jnp
from jax import lax
from jax.experimental import pallas as pl
from jax.experimental.pallas import tpu as pltpu
from jax.experimental.pallas import tpu_sc as plsc

N_NODES = 100000
H = 128

NC = 2
NS = 16
NW = NC * NS
CHUNK = 64
N_CHUNKS = 1600
N_PAD = CHUNK * N_CHUNKS
NSETS = 2
SLICES = (576, 768, 256)
BLK = 1024


def _sc_gather_body(npw, t_hbm, i0_hbm, i1_hbm, og0, og1,
                    i0v, i1v, bufs, semg, sems):
    cid = lax.axis_index("c")
    sid = lax.axis_index("s")
    wid = sid * NC + cid

    outs = (og0, og1)

    def drain_stores(s):
        for t in range(2):
            pltpu.make_async_copy(
                bufs.at[s, t], outs[t].at[pl.ds(0, CHUNK)],
                sems.at[s]).wait()

    def fire_gathers(s, j):
        return [
            pltpu.async_copy(t_hbm.at[i0v.at[j]], bufs.at[s, 0],
                             semg.at[s]),
            pltpu.async_copy(t_hbm.at[i1v.at[j]], bufs.at[s, 1],
                             semg.at[s]),
        ]

    def fire_stores(s, j):
        base = (wid * npw + j) * CHUNK
        for t in range(2):
            pltpu.async_copy(bufs.at[s, t], outs[t].at[pl.ds(base, CHUNK)],
                             sems.at[s])

    pltpu.sync_copy(i0_hbm.at[wid], i0v)
    pltpu.sync_copy(i1_hbm.at[wid], i1v)

    def group(q, carry):
        gs = []
        for s in range(NSETS):
            j = NSETS * q + s
            pl.when(q > 0)(lambda s=s: drain_stores(s))
            gs.append(fire_gathers(s, j))
        for s in range(NSETS):
            j = NSETS * q + s
            for g in gs[s]:
                g.wait()
            fire_stores(s, j)
        return carry

    lax.fori_loop(0, npw // NSETS, group, 0)
    for s in range(NSETS):
        drain_stores(s)


def _make_sc_gather(n_chunks):
    npw = n_chunks // NW
    assert npw * NW == n_chunks and npw % NSETS == 0
    mesh = plsc.VectorSubcoreMesh(core_axis_name="c", subcore_axis_name="s")
    row = jax.ShapeDtypeStruct((n_chunks * CHUNK, H), jnp.int32)

    def body(*args):
        return _sc_gather_body(npw, *args)

    return pl.kernel(
        body,
        mesh=mesh,
        out_type=(row, row),
        scratch_types=[
            pltpu.VMEM((npw, CHUNK), jnp.int32),
            pltpu.VMEM((npw, CHUNK), jnp.int32),
            pltpu.VMEM((NSETS, 2, CHUNK, H), jnp.int32),
            pltpu.SemaphoreType.DMA((NSETS,)),
            pltpu.SemaphoreType.DMA((NSETS,)),
        ],
    )


PBLK = 2048


def _pack_body(h_ref, c_ref, t_ref):
    def tobf(u):
        return (u + jnp.uint32(0x7FFF) + ((u >> 16) & jnp.uint32(1))) >> 16

    def packhalves(u):
        lo = tobf(u[:, :64])
        hi = tobf(u[:, 64:])
        return lo | (hi << 16)

    uh = lax.bitcast_convert_type(h_ref[...], jnp.uint32)
    uc = lax.bitcast_convert_type(c_ref[...], jnp.uint32)
    t = jnp.concatenate([packhalves(uh), packhalves(uc)], axis=1)
    t_ref[...] = lax.bitcast_convert_type(t, jnp.int32)


def _pack(h, c):
    n = h.shape[0]
    return pl.pallas_call(
        _pack_body,
        grid=(pl.cdiv(n, PBLK),),
        in_specs=[
            pl.BlockSpec((PBLK, H), lambda i: (i, 0)),
            pl.BlockSpec((PBLK, H), lambda i: (i, 0)),
        ],
        out_specs=pl.BlockSpec((PBLK, H), lambda i: (i, 0)),
        out_shape=jax.ShapeDtypeStruct((n, H), jnp.int32),
        compiler_params=pltpu.CompilerParams(
            dimension_semantics=("arbitrary",),
        ),
    )(h, c)


def _unpack(g_ref):
    u = lax.bitcast_convert_type(g_ref[...], jnp.uint32)

    def expand(half):
        lo = lax.bitcast_convert_type(half << 16, jnp.float32)
        hi = lax.bitcast_convert_type(half & jnp.uint32(0xFFFF0000),
                                      jnp.float32)
        return jnp.concatenate([lo, hi], axis=1)

    return expand(u[:, :64]), expand(u[:, 64:])


def _dense_body(x_ref, g0_ref, g1_ref,
                w_ref, u0_ref, u1_ref, b_ref, f0_ref, f1_ref, bf_ref,
                *refs):
    hout_ref, cout_ref = refs[-2:]
    x = x_ref[...]
    h0, c0 = _unpack(g0_ref)
    h1, c1 = _unpack(g1_ref)
    f32 = jnp.float32
    iou = (jnp.dot(x, w_ref[...], preferred_element_type=f32)
           + jnp.dot(h0, u0_ref[...], preferred_element_type=f32)
           + jnp.dot(h1, u1_ref[...], preferred_element_type=f32)
           + b_ref[...])
    fpre = (jnp.dot(h0, f0_ref[...], preferred_element_type=f32)
            + jnp.dot(h1, f1_ref[...], preferred_element_type=f32)
            + bf_ref[...])
    f = jax.nn.sigmoid(fpre)
    c_agg = f[:, :H] * c0 + f[:, H:] * c1
    i = jax.nn.sigmoid(iou[:, :H])
    o = jax.nn.sigmoid(iou[:, H:2 * H])
    u = jnp.tanh(iou[:, 2 * H:])
    c_new = i * u + c_agg
    hout_ref[...] = o * jnp.tanh(c_new)
    cout_ref[...] = c_new


def _dense_slice(x, g0, g1, weights, h_prev, c_prev, blk_off, n_blocks):
    W_iou, Um0, Um1, b_iou, Uf0, Uf1, bf = weights
    n = x.shape[0]
    full = lambda s: pl.BlockSpec(s, lambda i: (0, 0))
    off_spec = pl.BlockSpec((BLK, H), lambda i: (i + blk_off, 0))
    loc_spec = pl.BlockSpec((BLK, H), lambda i: (i, 0))
    in_specs = [
        off_spec, loc_spec, loc_spec,
        full((H, 3 * H)), full((H, 3 * H)), full((H, 3 * H)),
        full((1, 3 * H)),
        full((H, 2 * H)), full((H, 2 * H)), full((1, 2 * H)),
    ]
    args = [x, g0, g1, W_iou, Um0, Um1, b_iou, Uf0, Uf1, bf]
    aliases = {}
    if h_prev is not None:
        in_specs += [pl.BlockSpec(memory_space=pltpu.MemorySpace.HBM),
                     pl.BlockSpec(memory_space=pltpu.MemorySpace.HBM)]
        args += [h_prev, c_prev]
        aliases = {10: 0, 11: 1}
    return pl.pallas_call(
        _dense_body,
        grid=(n_blocks,),
        in_specs=in_specs,
        out_specs=[
            pl.BlockSpec((BLK, H), lambda i: (i + blk_off, 0)),
            pl.BlockSpec((BLK, H), lambda i: (i + blk_off, 0)),
        ],
        out_shape=[
            jax.ShapeDtypeStruct((n, H), jnp.float32),
            jax.ShapeDtypeStruct((n, H), jnp.float32),
        ],
        input_output_aliases=aliases,
        compiler_params=pltpu.CompilerParams(
            dimension_semantics=("arbitrary",),
        ),
    )(*args)


@jax.jit
def _run(x, h, c, idx0, idx1, W_iou, Um0, Um1, b_iou, Uf0, Uf1, bf):
    t = _pack(h, c)
    weights = (W_iou, Um0, Um1, b_iou, Uf0, Uf1, bf)
    n = x.shape[0]

    gathers = []
    off = 0
    for ns in SLICES:
        npw = ns // NW
        i0 = lax.dynamic_slice_in_dim(idx0, off * CHUNK, ns * CHUNK)
        i1 = lax.dynamic_slice_in_dim(idx1, off * CHUNK, ns * CHUNK)
        i0 = i0.reshape(NW, npw, CHUNK)
        i1 = i1.reshape(NW, npw, CHUNK)
        gathers.append(_make_sc_gather(ns)(t, i0, i1))
        off += ns

    h_out = c_out = None
    off = 0
    for k, ns in enumerate(SLICES):
        g0, g1 = gathers[k]
        lo = off * CHUNK
        hi = min((off + ns) * CHUNK, n)
        n_blocks = pl.cdiv(hi - lo, BLK)
        h_out, c_out = _dense_slice(x, g0, g1, weights, h_out, c_out,
                                    lo // BLK, n_blocks)
        off += ns
    return h_out, c_out


def kernel(x, h, c, child_idx, W_iou, Um0_iou, Um1_iou, b_iou, U_f_w, U_f_b):
    idx = child_idx.astype(jnp.int32)
    pad = N_PAD - N_NODES
    idx0 = jnp.pad(idx[:, 0], (0, pad))
    idx1 = jnp.pad(idx[:, 1], (0, pad))
    bf = U_f_b.reshape(1, 2 * H)
    return _run(x, h, c, idx0, idx1,
                W_iou, Um0_iou, Um1_iou, b_iou,
                U_f_w[:H, :], U_f_w[H:, :], bf)
